# baseline (speedup 1.0000x reference)
import functools

import jax
import jax.numpy as jnp
from jax import lax
from jax.experimental import pallas as pl
from jax.experimental.pallas import tpu as pltpu
from jax.experimental.pallas import tpu_sc as plsc

B_ROWS = 4096
NUM_FIELDS = 26
EMBED_DIM = 64
FIELD_SIZE = 40000
B = B_ROWS * NUM_FIELDS

NC = 2
NS = 16
NW = NC * NS
B_PER_W = B // NW
CHUNK = 832
N_CHUNKS = B_PER_W // CHUNK
LANES = 16
VECS_PER_CHUNK = CHUNK // LANES


def _body(x_hbm, table_hbm, out_hbm, xv, idxv, rowsv, sem):
    wid = lax.axis_index("s") * NC + lax.axis_index("c")
    lane = lax.iota(jnp.int32, LANES)

    def do_chunk(c, _):
        base = wid * B_PER_W + c * CHUNK
        pltpu.sync_copy(x_hbm.at[pl.ds(base, CHUNK)], xv)

        def add_offsets(j, _):
            pos = base + j * LANES + lane
            field = lax.rem(pos, NUM_FIELDS)
            idxv[pl.ds(j * LANES, LANES)] = (
                xv[pl.ds(j * LANES, LANES)] + field * FIELD_SIZE
            )
            return 0

        lax.fori_loop(0, VECS_PER_CHUNK, add_offsets, 0)
        pltpu.async_copy(table_hbm.at[idxv], rowsv, sem).wait()
        pltpu.sync_copy(rowsv, out_hbm.at[pl.ds(base, CHUNK)])
        return 0

    lax.fori_loop(0, N_CHUNKS, do_chunk, 0)


@jax.jit
def kernel(x, table):
    mesh = plsc.VectorSubcoreMesh(core_axis_name="c", subcore_axis_name="s")
    k = functools.partial(
        pl.kernel,
        mesh=mesh,
        out_type=jax.ShapeDtypeStruct((B, EMBED_DIM), jnp.float32),
        scratch_types=[
            pltpu.VMEM((CHUNK,), jnp.int32),
            pltpu.VMEM((CHUNK,), jnp.int32),
            pltpu.VMEM((CHUNK, EMBED_DIM), jnp.float32),
            pltpu.SemaphoreType.DMA,
        ],
        compiler_params=pltpu.CompilerParams(use_tc_tiling_on_sc=False),
    )(_body)
    out = k(x.reshape(-1), table)
    return out.reshape(B_ROWS, NUM_FIELDS, EMBED_DIM)
